# baseline (speedup 1.0000x reference)
import functools
import jax
import jax.numpy as jnp
from jax import lax
from jax.experimental import pallas as pl
from jax.experimental.pallas import tpu as pltpu
from jax.experimental.pallas import tpu_sc as plsc

NC = 2
NS = 16
NW = NC * NS
L = 16
CH = 32


def kernel(x, pe_weight):
    B, S, D = x.shape
    R = B * S
    rows_per_w = R // NW
    nch = rows_per_w // CH
    x2 = x.reshape(R, D)
    mesh = plsc.VectorSubcoreMesh(core_axis_name="c", subcore_axis_name="s")

    @functools.partial(
        pl.kernel,
        out_type=jax.ShapeDtypeStruct((R, D), jnp.float32),
        mesh=mesh,
        scratch_types=[
            pltpu.VMEM((CH, D), jnp.float32),
            pltpu.VMEM((CH, D), jnp.float32),
        ],
    )
    def k(x_hbm, pe_hbm, out_hbm, xbuf, pebuf):
        wid = lax.axis_index("s") * NC + lax.axis_index("c")
        base = wid * rows_per_w
        srow = lax.rem(base, S)
        for c in range(nch):
            r0 = base + c * CH
            p0 = srow + c * CH
            pltpu.sync_copy(x_hbm.at[pl.ds(r0, CH)], xbuf)
            pltpu.sync_copy(pe_hbm.at[pl.ds(p0, CH)], pebuf)

            def body(i, carry):
                for u in range(D // L):
                    sl = pl.ds(u * L, L)
                    xbuf[i, sl] = xbuf[i, sl] + pebuf[i, sl]
                return carry

            lax.fori_loop(0, CH, body, 0)
            pltpu.sync_copy(xbuf, out_hbm.at[pl.ds(r0, CH)])

    return k(x2, pe_weight).reshape(B, S, D)


# speedup vs baseline: 3.5691x; 3.5691x over previous
import jax
import jax.numpy as jnp
from jax.experimental import pallas as pl

S_BLK = 512


def _add_pe_kernel(x_ref, pe_ref, o_ref):
    o_ref[...] = x_ref[...] + pe_ref[...][None, :, :]


def kernel(x, pe_weight):
    B, S, D = x.shape
    grid = (S // S_BLK,)
    return pl.pallas_call(
        _add_pe_kernel,
        grid=grid,
        in_specs=[
            pl.BlockSpec((B, S_BLK, D), lambda i: (0, i, 0)),
            pl.BlockSpec((S_BLK, D), lambda i: (i, 0)),
        ],
        out_specs=pl.BlockSpec((B, S_BLK, D), lambda i: (0, i, 0)),
        out_shape=jax.ShapeDtypeStruct((B, S, D), x.dtype),
    )(x, pe_weight)
